# baseline (speedup 1.0000x reference)
import dataclasses
import functools

import jax
import jax.numpy as jnp
from jax import lax
from jax.experimental import pallas as pl
from jax.experimental.pallas import tpu as pltpu
from jax.experimental.pallas import tpu_sc as plsc

B, N, G = 64, 8192, 32 * 32 * 32
SIX_N = 6 * N
CHUNK = 2048


def _build_affine(output):
    planes = output[:, 0:3, :]
    nv = planes[..., 0:3]
    d = planes[..., 3]
    nn = jnp.sum(nv * nv, axis=-1)
    eye = jnp.eye(3, dtype=output.dtype)
    Mf = eye - 2.0 * nv[..., :, None] * nv[..., None, :] / nn[..., None, None]
    cf = -2.0 * d[..., None] * nv / nn[..., None]

    quats = output[:, 3:6, :]
    w = quats[..., 0]
    u = quats[..., 1:4]
    uu_sum = jnp.sum(u * u, axis=-1)
    nq = jnp.sqrt(w * w + uu_sum)
    uuT = u[..., :, None] * u[..., None, :]
    ux, uy, uz = u[..., 0], u[..., 1], u[..., 2]
    zz = jnp.zeros_like(ux)
    ucross = jnp.stack([
        jnp.stack([zz, -uz, uy], axis=-1),
        jnp.stack([uz, zz, -ux], axis=-1),
        jnp.stack([-uy, ux, zz], axis=-1),
    ], axis=-2)
    Mr = ((w * w - uu_sum)[..., None, None] * eye
          + 2.0 * uuT + 2.0 * w[..., None, None] * ucross) / nq[..., None, None]

    M = jnp.concatenate([Mf, Mr], axis=1)
    t = jnp.concatenate([cf, jnp.zeros_like(cf)], axis=1)
    Wfull = jnp.concatenate([M, t[..., None]], axis=-1)
    return Wfull.transpose(0, 2, 1, 3).reshape(B, 18, 4)


HB = B // 2


def _repack_body(x_ref, y_ref, z_ref, tx_ref, ty_ref, tz_ref):
    tx_ref[...] = x_ref[0, 0]
    ty_ref[...] = y_ref[0, 0]
    tz_ref[...] = z_ref[0, 0]


_repack_call = pl.pallas_call(
    _repack_body,
    grid=(HB,),
    in_specs=[pl.BlockSpec((1, 1, G), lambda b: (b, 0, 0))] * 3,
    out_specs=[pl.BlockSpec((G,), lambda b: (b,))] * 3,
    out_shape=[jax.ShapeDtypeStruct((HB * G,), jnp.float32)] * 3,
)


def _sym18(w_ref, pts_ref):
    wmat = w_ref[0]
    px = pts_ref[0, 0:1, :]
    py = pts_ref[0, 1:2, :]
    pz = pts_ref[0, 2:3, :]
    return (wmat[:, 0:1] * px + wmat[:, 1:2] * py
            + wmat[:, 2:3] * pz + wmat[:, 3:4])


def _lin_body(w_ref, pts_ref, *lin_refs):
    sym = _sym18(w_ref, pts_ref)
    ix = jnp.floor(jnp.clip(sym[0:6], 0.0, 31.0)).astype(jnp.int32)
    iy = jnp.floor(jnp.clip(sym[6:12], 0.0, 31.0)).astype(jnp.int32)
    iz = jnp.floor(jnp.clip(sym[12:18], 0.0, 31.0)).astype(jnp.int32)
    lin = jnp.clip(ix * 1024 + iy * 32 + iz, 0, G - 1)
    for s in range(6):
        lin_refs[s][...] = lin[s]


_lin_call = pl.pallas_call(
    _lin_body,
    grid=(HB,),
    in_specs=[
        pl.BlockSpec((1, 18, 4), lambda b: (b, 0, 0)),
        pl.BlockSpec((1, 3, N), lambda b: (b, 0, 0)),
    ],
    out_specs=[pl.BlockSpec((N,), lambda b: (b,))] * 6,
    out_shape=[jax.ShapeDtypeStruct((HB * N,), jnp.int32)] * 6,
)


_NCH = SIX_N // CHUNK


def _gather_body(tx_hbm, ty_hbm, tz_hbm,
                 l0, l1, l2, l3, l4, l5, cp_hbm,
                 tbx, tby, tbz, idxb0, idxb1, ox0, oy0, oz0, ox1, oy1, oz1,
                 stbl, sidx0, sidx1, sout0, sout1):
    lin_refs = (l0, l1, l2, l3, l4, l5)
    idxbufs = (idxb0, idxb1)
    outbufs = ((ox0, oy0, oz0), (ox1, oy1, oz1))
    sidx = (sidx0, sidx1)
    sout = (sout0, sout1)
    wid = lax.axis_index("s") * 2 + lax.axis_index("c")

    def start_idx(b, k):
        s, co = divmod(k, N // CHUNK)
        return pltpu.async_copy(
            lin_refs[s].at[pl.ds(b * N + co * CHUNK, CHUNK)],
            idxbufs[k % 2], sidx[k % 2])

    def start_out(b, k):
        s, co = divmod(k, N // CHUNK)
        base = (b * 18 + s) * N + co * CHUNK
        bufs = outbufs[k % 2]
        return [pltpu.async_copy(bufs[c], cp_hbm.at[pl.ds(base + c * 6 * N, CHUNK)],
                                 sout[k % 2]) for c in range(3)]

    if True:
        b = wid
        tdescs = [pltpu.async_copy(tx_hbm.at[pl.ds(b * G, G)], tbx, stbl),
                  pltpu.async_copy(ty_hbm.at[pl.ds(b * G, G)], tby, stbl),
                  pltpu.async_copy(tz_hbm.at[pl.ds(b * G, G)], tbz, stbl)]
        idescs = [None] * _NCH
        odescs = [None] * _NCH
        idescs[0] = start_idx(b, 0)
        idescs[1] = start_idx(b, 1)
        for t in tdescs:
            t.wait()

        for k in range(_NCH):
            idescs[k].wait()
            if k >= 2:
                for o in odescs[k - 2]:
                    o.wait()
            ib = idxbufs[k % 2]
            ox, oy, oz = outbufs[k % 2]

            @plsc.parallel_loop(0, CHUNK, step=16, unroll=4)
            def _(i):
                idxr = ib[pl.ds(i, 16)]
                ox[pl.ds(i, 16)] = plsc.load_gather(tbx, [idxr])
                oy[pl.ds(i, 16)] = plsc.load_gather(tby, [idxr])
                oz[pl.ds(i, 16)] = plsc.load_gather(tbz, [idxr])

            odescs[k] = start_out(b, k)
            if k + 2 < _NCH:
                idescs[k + 2] = start_idx(b, k + 2)

        for k in (_NCH - 2, _NCH - 1):
            for o in odescs[k]:
                o.wait()


@functools.cache
def _gather_call():
    cparams = pltpu.CompilerParams()
    if "needs_layout_passes" in pltpu.CompilerParams.__dataclass_fields__:
        cparams = dataclasses.replace(cparams, needs_layout_passes=False)
    return pl.kernel(
        _gather_body,
        compiler_params=cparams,
        out_type=jax.ShapeDtypeStruct((HB * 18 * N,), jnp.float32),
        mesh=plsc.VectorSubcoreMesh(core_axis_name="c", subcore_axis_name="s",
                                    num_cores=2, num_subcores=16),
        scratch_types=[
            pltpu.VMEM((G,), jnp.float32),
            pltpu.VMEM((G,), jnp.float32),
            pltpu.VMEM((G,), jnp.float32),
            pltpu.VMEM((CHUNK,), jnp.int32),
            pltpu.VMEM((CHUNK,), jnp.int32),
            pltpu.VMEM((CHUNK,), jnp.float32),
            pltpu.VMEM((CHUNK,), jnp.float32),
            pltpu.VMEM((CHUNK,), jnp.float32),
            pltpu.VMEM((CHUNK,), jnp.float32),
            pltpu.VMEM((CHUNK,), jnp.float32),
            pltpu.VMEM((CHUNK,), jnp.float32),
            pltpu.SemaphoreType.DMA,
            pltpu.SemaphoreType.DMA,
            pltpu.SemaphoreType.DMA,
            pltpu.SemaphoreType.DMA,
            pltpu.SemaphoreType.DMA,
        ],
    )


def _dist_body(w_ref, pts_ref, cp_ref, out_ref):
    b = pl.program_id(0)
    sym = _sym18(w_ref, pts_ref)
    cp = jnp.concatenate(
        [cp_ref[pl.ds(r * N, N)].reshape(1, N) for r in range(18)], axis=0)
    d = sym - cp
    sq = d * d
    ssq = sq[0:6] + sq[6:12] + sq[12:18]
    dist = jnp.sqrt(ssq)
    part = jnp.sum(dist) * (1.0 / (N * B))

    @pl.when(b == 0)
    def _():
        out_ref[...] = jnp.zeros_like(out_ref)

    out_ref[...] += part


_dist_call = pl.pallas_call(
    _dist_body,
    grid=(HB,),
    in_specs=[
        pl.BlockSpec((1, 18, 4), lambda b: (b, 0, 0)),
        pl.BlockSpec((1, 3, N), lambda b: (b, 0, 0)),
        pl.BlockSpec((18 * N,), lambda b: (b,)),
    ],
    out_specs=pl.BlockSpec((1, 1), lambda b: (0, 0)),
    out_shape=jax.ShapeDtypeStruct((1, 1), jnp.float32),
)


def kernel(output, points, closest):
    w = _build_affine(output)
    losses = []
    cps = []
    hw = []
    for h in range(2):
        sl = slice(h * HB, (h + 1) * HB)
        cl = closest[sl]
        planes = (cl[:, None, :, 0], cl[:, None, :, 1], cl[:, None, :, 2])
        pts_h = points[sl].transpose(0, 2, 1)
        w_h = w[sl]
        tabs = _repack_call(*planes)
        lins = _lin_call(w_h, pts_h)
        cps.append(_gather_call()(*tabs, *lins))
        hw.append((w_h, pts_h))
    for h in range(2):
        w_h, pts_h = hw[h]
        losses.append(_dist_call(w_h, pts_h, cps[h]))
    return (losses[0] + losses[1]).reshape(1)


# speedup vs baseline: 1.0186x; 1.0186x over previous
import dataclasses
import functools

import jax
import jax.numpy as jnp
from jax import lax
from jax.experimental import pallas as pl
from jax.experimental.pallas import tpu as pltpu
from jax.experimental.pallas import tpu_sc as plsc

B, N, G = 64, 8192, 32 * 32 * 32
SIX_N = 6 * N
CHUNK = 2048


def _build_affine(output):
    planes = output[:, 0:3, :]
    nv = planes[..., 0:3]
    d = planes[..., 3]
    nn = jnp.sum(nv * nv, axis=-1)
    eye = jnp.eye(3, dtype=output.dtype)
    Mf = eye - 2.0 * nv[..., :, None] * nv[..., None, :] / nn[..., None, None]
    cf = -2.0 * d[..., None] * nv / nn[..., None]

    quats = output[:, 3:6, :]
    w = quats[..., 0]
    u = quats[..., 1:4]
    uu_sum = jnp.sum(u * u, axis=-1)
    nq = jnp.sqrt(w * w + uu_sum)
    uuT = u[..., :, None] * u[..., None, :]
    ux, uy, uz = u[..., 0], u[..., 1], u[..., 2]
    zz = jnp.zeros_like(ux)
    ucross = jnp.stack([
        jnp.stack([zz, -uz, uy], axis=-1),
        jnp.stack([uz, zz, -ux], axis=-1),
        jnp.stack([-uy, ux, zz], axis=-1),
    ], axis=-2)
    Mr = ((w * w - uu_sum)[..., None, None] * eye
          + 2.0 * uuT + 2.0 * w[..., None, None] * ucross) / nq[..., None, None]

    M = jnp.concatenate([Mf, Mr], axis=1)
    t = jnp.concatenate([cf, jnp.zeros_like(cf)], axis=1)
    Wfull = jnp.concatenate([M, t[..., None]], axis=-1)
    return Wfull.transpose(0, 2, 1, 3).reshape(B, 18, 4)


HB = B // 4


def _repack_body(x_ref, y_ref, z_ref, tx_ref, ty_ref, tz_ref):
    tx_ref[...] = x_ref[0, 0]
    ty_ref[...] = y_ref[0, 0]
    tz_ref[...] = z_ref[0, 0]


_repack_call = pl.pallas_call(
    _repack_body,
    grid=(HB,),
    in_specs=[pl.BlockSpec((1, 1, G), lambda b: (b, 0, 0))] * 3,
    out_specs=[pl.BlockSpec((G,), lambda b: (b,))] * 3,
    out_shape=[jax.ShapeDtypeStruct((HB * G,), jnp.float32)] * 3,
)


def _sym18(w_ref, pts_ref):
    wmat = w_ref[0]
    px = pts_ref[0, 0:1, :]
    py = pts_ref[0, 1:2, :]
    pz = pts_ref[0, 2:3, :]
    return (wmat[:, 0:1] * px + wmat[:, 1:2] * py
            + wmat[:, 2:3] * pz + wmat[:, 3:4])


def _lin_body(w_ref, pts_ref, *lin_refs):
    sym = _sym18(w_ref, pts_ref)
    ix = jnp.floor(jnp.clip(sym[0:6], 0.0, 31.0)).astype(jnp.int32)
    iy = jnp.floor(jnp.clip(sym[6:12], 0.0, 31.0)).astype(jnp.int32)
    iz = jnp.floor(jnp.clip(sym[12:18], 0.0, 31.0)).astype(jnp.int32)
    lin = jnp.clip(ix * 1024 + iy * 32 + iz, 0, G - 1)
    for s in range(6):
        lin_refs[s][...] = lin[s]


_lin_call = pl.pallas_call(
    _lin_body,
    grid=(HB,),
    in_specs=[
        pl.BlockSpec((1, 18, 4), lambda b: (b, 0, 0)),
        pl.BlockSpec((1, 3, N), lambda b: (b, 0, 0)),
    ],
    out_specs=[pl.BlockSpec((N,), lambda b: (b,))] * 6,
    out_shape=[jax.ShapeDtypeStruct((HB * N,), jnp.int32)] * 6,
)


_NCH = SIX_N // CHUNK


def _gather_body(tx_hbm, ty_hbm, tz_hbm,
                 l0, l1, l2, l3, l4, l5, cp_hbm,
                 tbx, tby, tbz, idxb0, idxb1, ox0, oy0, oz0, ox1, oy1, oz1,
                 stbl, sidx0, sidx1, sout0, sout1):
    lin_refs = (l0, l1, l2, l3, l4, l5)
    idxbufs = (idxb0, idxb1)
    outbufs = ((ox0, oy0, oz0), (ox1, oy1, oz1))
    sidx = (sidx0, sidx1)
    sout = (sout0, sout1)

    def start_idx(b, k):
        s, co = divmod(k, N // CHUNK)
        return pltpu.async_copy(
            lin_refs[s].at[pl.ds(b * N + co * CHUNK, CHUNK)],
            idxbufs[k % 2], sidx[k % 2])

    def start_out(b, k):
        s, co = divmod(k, N // CHUNK)
        base = (b * 18 + s) * N + co * CHUNK
        bufs = outbufs[k % 2]
        return [pltpu.async_copy(bufs[c], cp_hbm.at[pl.ds(base + c * 6 * N, CHUNK)],
                                 sout[k % 2]) for c in range(3)]

    b = lax.axis_index("s")
    parity = lax.axis_index("c")

    tdescs = [pltpu.async_copy(tx_hbm.at[pl.ds(b * G, G)], tbx, stbl),
              pltpu.async_copy(ty_hbm.at[pl.ds(b * G, G)], tby, stbl),
              pltpu.async_copy(tz_hbm.at[pl.ds(b * G, G)], tbz, stbl)]
    for t in tdescs:
        t.wait()

    for p in range(2):
        @pl.when(parity == p)
        def _(p=p):
            k0 = p * (_NCH // 2)
            kn = _NCH // 2
            idescs = {}
            odescs = {}
            idescs[k0] = start_idx(b, k0)
            idescs[k0 + 1] = start_idx(b, k0 + 1)
            for k in range(k0, k0 + kn):
                idescs[k].wait()
                if k - 2 >= k0:
                    for o in odescs[k - 2]:
                        o.wait()
                ib = idxbufs[k % 2]
                ox, oy, oz = outbufs[k % 2]

                @plsc.parallel_loop(0, CHUNK, step=16, unroll=4)
                def _(i):
                    idxr = ib[pl.ds(i, 16)]
                    ox[pl.ds(i, 16)] = plsc.load_gather(tbx, [idxr])
                    oy[pl.ds(i, 16)] = plsc.load_gather(tby, [idxr])
                    oz[pl.ds(i, 16)] = plsc.load_gather(tbz, [idxr])

                odescs[k] = start_out(b, k)
                if k + 2 < k0 + kn:
                    idescs[k + 2] = start_idx(b, k + 2)

            for k in (k0 + kn - 2, k0 + kn - 1):
                for o in odescs[k]:
                    o.wait()


@functools.cache
def _gather_call():
    cparams = pltpu.CompilerParams()
    if "needs_layout_passes" in pltpu.CompilerParams.__dataclass_fields__:
        cparams = dataclasses.replace(cparams, needs_layout_passes=False)
    return pl.kernel(
        _gather_body,
        compiler_params=cparams,
        out_type=jax.ShapeDtypeStruct((HB * 18 * N,), jnp.float32),
        mesh=plsc.VectorSubcoreMesh(core_axis_name="c", subcore_axis_name="s",
                                    num_cores=2, num_subcores=16),
        scratch_types=[
            pltpu.VMEM((G,), jnp.float32),
            pltpu.VMEM((G,), jnp.float32),
            pltpu.VMEM((G,), jnp.float32),
            pltpu.VMEM((CHUNK,), jnp.int32),
            pltpu.VMEM((CHUNK,), jnp.int32),
            pltpu.VMEM((CHUNK,), jnp.float32),
            pltpu.VMEM((CHUNK,), jnp.float32),
            pltpu.VMEM((CHUNK,), jnp.float32),
            pltpu.VMEM((CHUNK,), jnp.float32),
            pltpu.VMEM((CHUNK,), jnp.float32),
            pltpu.VMEM((CHUNK,), jnp.float32),
            pltpu.SemaphoreType.DMA,
            pltpu.SemaphoreType.DMA,
            pltpu.SemaphoreType.DMA,
            pltpu.SemaphoreType.DMA,
            pltpu.SemaphoreType.DMA,
        ],
    )


def _dist_body(w_ref, pts_ref, cp_ref, out_ref):
    b = pl.program_id(0)
    sym = _sym18(w_ref, pts_ref)
    cp = jnp.concatenate(
        [cp_ref[pl.ds(r * N, N)].reshape(1, N) for r in range(18)], axis=0)
    d = sym - cp
    sq = d * d
    ssq = sq[0:6] + sq[6:12] + sq[12:18]
    dist = jnp.sqrt(ssq)
    part = jnp.sum(dist) * (1.0 / (N * B))

    @pl.when(b == 0)
    def _():
        out_ref[...] = jnp.zeros_like(out_ref)

    out_ref[...] += part


_dist_call = pl.pallas_call(
    _dist_body,
    grid=(HB,),
    in_specs=[
        pl.BlockSpec((1, 18, 4), lambda b: (b, 0, 0)),
        pl.BlockSpec((1, 3, N), lambda b: (b, 0, 0)),
        pl.BlockSpec((18 * N,), lambda b: (b,)),
    ],
    out_specs=pl.BlockSpec((1, 1), lambda b: (0, 0)),
    out_shape=jax.ShapeDtypeStruct((1, 1), jnp.float32),
)


def kernel(output, points, closest):
    w = _build_affine(output)
    cps = []
    hw = []
    for h in range(4):
        sl = slice(h * HB, (h + 1) * HB)
        cl = closest[sl]
        planes = (cl[:, None, :, 0], cl[:, None, :, 1], cl[:, None, :, 2])
        pts_h = points[sl].transpose(0, 2, 1)
        w_h = w[sl]
        tabs = _repack_call(*planes)
        lins = _lin_call(w_h, pts_h)
        cps.append(_gather_call()(*tabs, *lins))
        hw.append((w_h, pts_h))
    losses = [_dist_call(w_h, pts_h, cp) for (w_h, pts_h), cp in zip(hw, cps)]
    return (losses[0] + losses[1] + losses[2] + losses[3]).reshape(1)
